# Initial kernel scaffold; baseline (speedup 1.0000x reference)
import functools

import jax
import jax.numpy as jnp
from jax import lax
from jax.experimental import pallas as pl
from jax.experimental.pallas import tpu as pltpu
from jax.experimental.pallas import tpu_sc as plsc

BATCH = 16384
FIELDS = 26
EMBED_DIM = 32
N_ROWS = BATCH * FIELDS

NUM_CORES = 2
NUM_SUBCORES = 16
NW = NUM_CORES * NUM_SUBCORES
B_PER_W = N_ROWS // NW
G = 128
K = B_PER_W // G

_mesh = plsc.VectorSubcoreMesh(core_axis_name="c", subcore_axis_name="s")


@functools.partial(
    pl.kernel,
    mesh=_mesh,
    out_type=jax.ShapeDtypeStruct((N_ROWS, EMBED_DIM), jnp.float32),
    scratch_types=[
        pltpu.VMEM((K, G), jnp.int32),
        pltpu.VMEM((G, EMBED_DIM), jnp.float32),
        pltpu.SemaphoreType.DMA,
    ],
)
def _emb_gather(idx_hbm, table_hbm, out_hbm, idx_v, rows_v, sem):
    wid = lax.axis_index("s") * NUM_CORES + lax.axis_index("c")
    base = wid * B_PER_W
    pltpu.sync_copy(idx_hbm.at[wid], idx_v)

    def body(j, carry):
        pltpu.async_copy(table_hbm.at[idx_v.at[j]], rows_v, sem).wait()
        pltpu.sync_copy(rows_v, out_hbm.at[pl.ds(base + j * G, G)])
        return carry

    lax.fori_loop(0, K, body, 0, unroll=False)


def kernel(x, table):
    idx = x.astype(jnp.int32).reshape(NW, K, G)
    out = _emb_gather(idx, table)
    return out.reshape(BATCH, FIELDS, EMBED_DIM)


# speedup vs baseline: 1.4378x; 1.4378x over previous
import functools

import jax
import jax.numpy as jnp
from jax import lax
from jax.experimental import pallas as pl
from jax.experimental.pallas import tpu as pltpu
from jax.experimental.pallas import tpu_sc as plsc

BATCH = 16384
FIELDS = 26
EMBED_DIM = 32
N_ROWS = BATCH * FIELDS

NUM_CORES = 2
NUM_SUBCORES = 16
NW = NUM_CORES * NUM_SUBCORES
B_PER_W = N_ROWS // NW
G = 128
K = B_PER_W // G

_mesh = plsc.VectorSubcoreMesh(core_axis_name="c", subcore_axis_name="s")


@functools.partial(
    pl.kernel,
    mesh=_mesh,
    compiler_params=pltpu.CompilerParams(use_tc_tiling_on_sc=False),
    out_type=jax.ShapeDtypeStruct((N_ROWS, EMBED_DIM), jnp.float32),
    scratch_types=[
        pltpu.VMEM((K, G), jnp.int32),
        pltpu.VMEM((G, EMBED_DIM), jnp.float32),
        pltpu.SemaphoreType.DMA,
    ],
)
def _emb_gather(idx_hbm, table_hbm, out_hbm, idx_v, rows_v, sem):
    wid = lax.axis_index("s") * NUM_CORES + lax.axis_index("c")
    base = wid * B_PER_W
    pltpu.sync_copy(idx_hbm.at[wid], idx_v)

    def body(j, carry):
        pltpu.async_copy(table_hbm.at[idx_v.at[j]], rows_v, sem).wait()
        pltpu.sync_copy(rows_v, out_hbm.at[pl.ds(base + j * G, G)])
        return carry

    lax.fori_loop(0, K, body, 0, unroll=False)


def kernel(x, table):
    idx = x.astype(jnp.int32).reshape(NW, K, G)
    out = _emb_gather(idx, table)
    return out.reshape(BATCH, FIELDS, EMBED_DIM)


# speedup vs baseline: 1.5615x; 1.0861x over previous
import functools

import jax
import jax.numpy as jnp
from jax import lax
from jax.experimental import pallas as pl
from jax.experimental.pallas import tpu as pltpu
from jax.experimental.pallas import tpu_sc as plsc

BATCH = 16384
FIELDS = 26
EMBED_DIM = 32
N_ROWS = BATCH * FIELDS

NUM_CORES = 2
NUM_SUBCORES = 16
NW = NUM_CORES * NUM_SUBCORES
B_PER_W = N_ROWS // NW
G = 128
K = B_PER_W // G

_mesh = plsc.VectorSubcoreMesh(core_axis_name="c", subcore_axis_name="s")


NBUF = 4
T = K // NBUF


@functools.partial(
    pl.kernel,
    mesh=_mesh,
    compiler_params=pltpu.CompilerParams(use_tc_tiling_on_sc=False),
    out_type=jax.ShapeDtypeStruct((N_ROWS, EMBED_DIM), jnp.float32),
    scratch_types=[
        pltpu.VMEM((K, G), jnp.int32),
        pltpu.VMEM((NBUF, G, EMBED_DIM), jnp.float32),
        pltpu.SemaphoreType.DMA,
        pltpu.SemaphoreType.DMA,
        pltpu.SemaphoreType.DMA,
        pltpu.SemaphoreType.DMA,
        pltpu.SemaphoreType.DMA,
        pltpu.SemaphoreType.DMA,
        pltpu.SemaphoreType.DMA,
        pltpu.SemaphoreType.DMA,
    ],
)
def _emb_gather(idx_hbm, table_hbm, out_hbm, idx_v, rows_v,
                g0, g1, g2, g3, s0, s1, s2, s3):
    gsems = (g0, g1, g2, g3)
    ssems = (s0, s1, s2, s3)
    wid = lax.axis_index("s") * NUM_CORES + lax.axis_index("c")
    base = wid * B_PER_W
    pltpu.sync_copy(idx_hbm.at[wid], idx_v)

    for b in range(NBUF):
        pltpu.async_copy(table_hbm.at[idx_v.at[b]], rows_v.at[b], gsems[b])

    def outer(t, carry):
        j0 = t * NBUF
        for b in range(NBUF):
            pltpu.make_async_copy(
                table_hbm.at[pl.ds(0, G)], rows_v.at[b], gsems[b]).wait()
            pltpu.async_copy(
                rows_v.at[b],
                out_hbm.at[pl.ds(base + (j0 + b) * G, G)],
                ssems[b])
        for b in range(NBUF):
            pltpu.make_async_copy(
                rows_v.at[b], out_hbm.at[pl.ds(base, G)], ssems[b]).wait()

            @pl.when(t < T - 1)
            def _():
                pltpu.async_copy(
                    table_hbm.at[idx_v.at[j0 + NBUF + b]],
                    rows_v.at[b], gsems[b])
        return carry

    lax.fori_loop(0, T, outer, 0, unroll=False)


def kernel(x, table):
    idx = x.astype(jnp.int32).reshape(NW, K, G)
    out = _emb_gather(idx, table)
    return out.reshape(BATCH, FIELDS, EMBED_DIM)
